# baseline (speedup 1.0000x reference)
import functools

import jax
import jax.numpy as jnp
from jax import lax
from jax.experimental import pallas as pl
from jax.experimental.pallas import tpu as pltpu
from jax.experimental.pallas import tpu_sc as plsc

S, D, F, O, E, K, T = 2048, 1024, 2048, 1024, 8, 2, 256
NT = S * K // T + E
NROWS = NT * T
NSC, NSUB = 2, 16
NW = NSC * NSUB
TPW = S // NW


def _k0_body(x_ref, gw_ref, gb_ref, w_out, pos_out, te_out):
    x = x_ref[...]
    f32 = jnp.float32
    xh = x.astype(jnp.bfloat16)
    gh = gw_ref[...].astype(jnp.bfloat16)
    logits = jnp.dot(xh, gh, preferred_element_type=f32) + gb_ref[...]
    m = jnp.max(logits, axis=-1, keepdims=True)
    ex = jnp.exp(logits - m)
    w = ex / jnp.sum(ex, axis=-1, keepdims=True)
    lane = jax.lax.broadcasted_iota(jnp.int32, w.shape, 1)
    m1 = jnp.max(w, axis=-1, keepdims=True)
    i1 = jnp.min(jnp.where(w == m1, lane, E), axis=-1, keepdims=True)
    sel1 = lane == i1
    w_rest = jnp.where(sel1, -1.0, w)
    m2 = jnp.max(w_rest, axis=-1, keepdims=True)
    i2 = jnp.min(jnp.where(w_rest == m2, lane, E), axis=-1, keepdims=True)
    sel2 = lane == i2

    oh1 = jnp.where(sel1, 1.0, 0.0)
    oh2 = jnp.where(sel2, 1.0, 0.0)
    r_iota = jax.lax.broadcasted_iota(jnp.int32, (S, S), 0)
    c_iota = jax.lax.broadcasted_iota(jnp.int32, (S, S), 1)
    Ls = jnp.where(c_iota < r_iota, 1.0, 0.0).astype(jnp.bfloat16)
    cum1 = jnp.dot(Ls, oh1.astype(jnp.bfloat16), preferred_element_type=f32)
    cum2 = jnp.dot(Ls, oh2.astype(jnp.bfloat16), preferred_element_type=f32)
    r1 = jnp.sum(cum1 * oh1, axis=-1, keepdims=True)
    r2 = jnp.sum(cum2 * oh2, axis=-1, keepdims=True)
    cnt1 = jnp.sum(oh1, axis=0, keepdims=True)
    cnt2 = jnp.sum(oh2, axis=0, keepdims=True)
    cap = jnp.floor((cnt1 + cnt2 + (T - 1)) / T) * T

    lane8 = jax.lax.broadcasted_iota(jnp.int32, (1, E), 1)
    off = jnp.zeros((1, E), f32)
    for j in range(1, E):
        cap_j = jnp.sum(jnp.where(lane8 == j - 1, cap, 0.0), axis=-1,
                        keepdims=True)
        off = off + jnp.where(lane8 >= j, cap_j, 0.0)

    pos1 = jnp.sum(oh1 * off, axis=-1, keepdims=True) + r1
    pos2 = (jnp.sum(oh2 * (off + cnt1), axis=-1, keepdims=True) + r2)

    lane32 = jax.lax.broadcasted_iota(jnp.int32, (1, 32), 1)
    tstart = (lane32 * T).astype(f32)
    te = jnp.zeros((1, 32), f32)
    for e in range(E):
        off_e = jnp.sum(jnp.where(lane8 == e, off, 0.0), axis=-1,
                        keepdims=True)
        cap_e = jnp.sum(jnp.where(lane8 == e, cap, 0.0), axis=-1,
                        keepdims=True)
        inside = (tstart >= off_e) & (tstart < off_e + cap_e)
        te = te + jnp.where(inside, float(e), 0.0)

    w_out[...] = jnp.concatenate([m1, m2], axis=1)
    pos_out[...] = jnp.concatenate([pos1, pos2], axis=1)
    te_out[...] = te


def _k0_call(x2d, gate_W, gate_b2d):
    return pl.pallas_call(
        _k0_body,
        grid=(1,),
        in_specs=[
            pl.BlockSpec((S, D), lambda i: (0, 0)),
            pl.BlockSpec((D, E), lambda i: (0, 0)),
            pl.BlockSpec((1, E), lambda i: (0, 0)),
        ],
        out_specs=[
            pl.BlockSpec((S, K), lambda i: (0, 0)),
            pl.BlockSpec((S, K), lambda i: (0, 0)),
            pl.BlockSpec((1, 32), lambda i: (0, 0)),
        ],
        out_shape=[
            jax.ShapeDtypeStruct((S, K), jnp.float32),
            jax.ShapeDtypeStruct((S, K), jnp.float32),
            jax.ShapeDtypeStruct((1, 32), jnp.float32),
        ],
    )(x2d, gate_W, gate_b2d)


def _sc1_body(xb_hbm, p1_hbm, p2_hbm, xs_hbm, rows_v, p1_v, p2_v, sem1,
              sem2):
    wid = lax.axis_index("s") * NSC + lax.axis_index("c")
    base = wid * TPW
    pltpu.sync_copy(xb_hbm.at[pl.ds(base, TPW)], rows_v)
    pltpu.sync_copy(p1_hbm.at[wid], p1_v)
    pltpu.sync_copy(p2_hbm.at[wid], p2_v)
    c1 = pltpu.async_copy(rows_v, xs_hbm.at[p1_v], sem1)
    c2 = pltpu.async_copy(rows_v, xs_hbm.at[p2_v], sem2)
    c1.wait()
    c2.wait()


def _sc1_call(xb3d, pos1, pos2):
    mesh = plsc.VectorSubcoreMesh(core_axis_name="c", subcore_axis_name="s")
    kern = functools.partial(
        pl.kernel, mesh=mesh,
        out_type=jax.ShapeDtypeStruct((NROWS, 4, 128), jnp.int32),
        scratch_types=[
            pltpu.VMEM((TPW, 4, 128), jnp.int32),
            pltpu.VMEM((TPW,), jnp.int32),
            pltpu.VMEM((TPW,), jnp.int32),
            pltpu.SemaphoreType.DMA,
            pltpu.SemaphoreType.DMA,
        ],
    )(_sc1_body)
    return kern(xb3d, pos1, pos2)


def _k1_body(x_ref, w1_ref, b1_ref, w2_ref, b2_ref, out_ref, *, tile,
             n_shared):
    e = pl.program_id(0)
    t = pl.program_id(1)
    rows = pl.ds(t * tile, tile)
    xb = x_ref[...].astype(jnp.bfloat16)
    FC = F // 4
    y = b2_ref[0]
    for f in range(4):
        fs = pl.ds(f * FC, FC)
        h = jnp.dot(xb, w1_ref[0, :, fs], preferred_element_type=jnp.float32)
        h = jnp.maximum(h + b1_ref[0, :, fs], 0.0).astype(jnp.bfloat16)
        y = y + jnp.dot(h, w2_ref[0, fs, :],
                        preferred_element_type=jnp.float32)
    y = y * (1.0 / n_shared)

    @pl.when(e == 0)
    def _init():
        out_ref[rows, :] = y

    @pl.when(e > 0)
    def _accum():
        out_ref[rows, :] += y


def _k1_call(x2d, sW1, sb1, sW2, sb2, tile=512):
    ns = sW1.shape[0]
    nt = S // tile
    body = functools.partial(_k1_body, tile=tile, n_shared=ns)
    return pl.pallas_call(
        body,
        grid=(ns, nt),
        in_specs=[
            pl.BlockSpec((tile, D), lambda e, t: (t, 0)),
            pl.BlockSpec((1, D, F), lambda e, t: (e, 0, 0)),
            pl.BlockSpec((1, 1, F), lambda e, t: (e, 0, 0)),
            pl.BlockSpec((1, F, O), lambda e, t: (e, 0, 0)),
            pl.BlockSpec((1, 1, O), lambda e, t: (e, 0, 0)),
        ],
        out_specs=pl.BlockSpec((S, O), lambda e, t: (0, 0)),
        out_shape=jax.ShapeDtypeStruct((S, O), jnp.float32),
    )(x2d, sW1, sb1, sW2, sb2)


def _k2_body(te_ref, xs_ref, w1_ref, b1_ref, w2_ref, b2_ref, y_ref):
    xb = xs_ref[...]
    FC = F // 4
    y = b2_ref[0]
    for f in range(4):
        fs = pl.ds(f * FC, FC)
        h = jnp.dot(xb, w1_ref[0, :, fs], preferred_element_type=jnp.float32)
        h = jnp.maximum(h + b1_ref[0, :, fs], 0.0).astype(jnp.bfloat16)
        y = y + jnp.dot(h, w2_ref[0, fs, :],
                        preferred_element_type=jnp.float32)
    y_ref[...] = y


def _k2_call(te, xs_bf2d, rW1, rb1, rW2, rb2):
    grid_spec = pltpu.PrefetchScalarGridSpec(
        num_scalar_prefetch=1,
        grid=(NT,),
        in_specs=[
            pl.BlockSpec((T, D), lambda t, te: (t, 0)),
            pl.BlockSpec((1, D, F), lambda t, te: (te[t], 0, 0)),
            pl.BlockSpec((1, 1, F), lambda t, te: (te[t], 0, 0)),
            pl.BlockSpec((1, F, O), lambda t, te: (te[t], 0, 0)),
            pl.BlockSpec((1, 1, O), lambda t, te: (te[t], 0, 0)),
        ],
        out_specs=pl.BlockSpec((T, O), lambda t, te: (t, 0)),
    )
    return pl.pallas_call(
        _k2_body,
        grid_spec=grid_spec,
        out_shape=jax.ShapeDtypeStruct((NROWS, O), jnp.float32),
    )(te, xs_bf2d, rW1, rb1, rW2, rb2)


_CH = 32


def _sc2_body(sh_hbm, ys_hbm, p1_hbm, p2_hbm, w1_hbm, w2_hbm, out_hbm,
              sh_v, y1_v, y2_v, p1_v, p2_v, w1_s, w2_s, sem0, sem1, sem2):
    wid = lax.axis_index("s") * NSC + lax.axis_index("c")
    base = wid * TPW
    pltpu.sync_copy(p1_hbm.at[wid], p1_v)
    pltpu.sync_copy(p2_hbm.at[wid], p2_v)
    pltpu.sync_copy(w1_hbm.at[wid], w1_s)
    pltpu.sync_copy(w2_hbm.at[wid], w2_s)
    for c in range(TPW // _CH):
        rows = pl.ds(base + c * _CH, _CH)
        c0 = pltpu.async_copy(sh_hbm.at[rows], sh_v, sem0)
        c1 = pltpu.async_copy(ys_hbm.at[p1_v.at[pl.ds(c * _CH, _CH)]], y1_v,
                              sem1)
        c2 = pltpu.async_copy(ys_hbm.at[p2_v.at[pl.ds(c * _CH, _CH)]], y2_v,
                              sem2)
        c0.wait()
        c1.wait()
        c2.wait()

        def grp(g, _):
            w1v = w1_s[pl.ds(c * _CH + g * 16, 16)]
            w2v = w2_s[pl.ds(c * _CH + g * 16, 16)]
            for i in range(16):
                w1 = w1v[i]
                w2 = w2v[i]
                tok = g * 16 + i

                def sub(j, _, tok=tok, w1=w1, w2=w2):
                    for l in range(8):
                        sl = pl.ds(l * 16, 16)
                        sh_v[tok, j, sl] = (sh_v[tok, j, sl]
                                            + w1 * y1_v[tok, j, sl]
                                            + w2 * y2_v[tok, j, sl])
                    return 0

                lax.fori_loop(0, 8, sub, 0, unroll=False)
            return 0

        lax.fori_loop(0, _CH // 16, grp, 0, unroll=False)
        pltpu.sync_copy(sh_v, out_hbm.at[rows])


def _sc2_call(sh3d, ys3d, pos1, pos2, w1, w2):
    mesh = plsc.VectorSubcoreMesh(core_axis_name="c", subcore_axis_name="s")
    kern = functools.partial(
        pl.kernel, mesh=mesh,
        out_type=jax.ShapeDtypeStruct((S, 8, 128), jnp.float32),
        scratch_types=[
            pltpu.VMEM((_CH, 8, 128), jnp.float32),
            pltpu.VMEM((_CH, 8, 128), jnp.float32),
            pltpu.VMEM((_CH, 8, 128), jnp.float32),
            pltpu.VMEM((TPW,), jnp.int32),
            pltpu.VMEM((TPW,), jnp.int32),
            pltpu.VMEM((TPW,), jnp.float32),
            pltpu.VMEM((TPW,), jnp.float32),
            pltpu.SemaphoreType.DMA,
            pltpu.SemaphoreType.DMA,
            pltpu.SemaphoreType.DMA,
        ],
    )(_sc2_body)
    return kern(sh3d, ys3d, pos1, pos2, w1, w2)


def kernel(x, gate_W, gate_b, shared_W1, shared_b1, shared_W2, shared_b2,
           routed_W1, routed_b1, routed_W2, routed_b2):
    B = x.shape[0]
    x2d = x.reshape(S, D)

    w_out, pos_out, te_out = _k0_call(x2d, gate_W, gate_b.reshape(1, E))

    te = te_out.reshape(32).astype(jnp.int32)
    posT = pos_out.T.astype(jnp.int32).reshape(K, NW, TPW)
    wT = w_out.T.reshape(K, NW, TPW)

    xb = x2d.astype(jnp.bfloat16)
    xi = jax.lax.bitcast_convert_type(xb.reshape(S, D // 2, 2),
                                      jnp.int32).reshape(S, 4, 128)
    xs_i = _sc1_call(xi, posT[0], posT[1])
    xs_bf = jax.lax.bitcast_convert_type(
        xs_i.reshape(NROWS, D // 2), jnp.bfloat16).reshape(NROWS, D)

    shared = _k1_call(x2d, shared_W1.astype(jnp.bfloat16),
                      shared_b1.reshape(-1, 1, F),
                      shared_W2.astype(jnp.bfloat16),
                      shared_b2.reshape(-1, 1, O))

    ys = _k2_call(te, xs_bf,
                  routed_W1.astype(jnp.bfloat16), routed_b1.reshape(E, 1, F),
                  routed_W2.astype(jnp.bfloat16), routed_b2.reshape(E, 1, O))

    out = _sc2_call(shared.reshape(S, 8, 128), ys.reshape(NROWS, 8, 128),
                    posT[0], posT[1], wT[0], wT[1])
    return out.reshape(B, S, O)


# speedup vs baseline: 1.8045x; 1.8045x over previous
import functools

import jax
import jax.numpy as jnp
from jax import lax
from jax.experimental import pallas as pl
from jax.experimental.pallas import tpu as pltpu
from jax.experimental.pallas import tpu_sc as plsc

S, D, F, O, E, K, T = 2048, 1024, 2048, 1024, 8, 2, 256
NT = S * K // T + E
NROWS = NT * T
NSC, NSUB = 2, 16
NW = NSC * NSUB
TPW = S // NW


def _k0_body(x_ref, gw_ref, gb_ref, w_out, pos_out, te_out, xp_out):
    x = x_ref[...]
    f32 = jnp.float32
    xh = x.astype(jnp.bfloat16)
    gh = gw_ref[...].astype(jnp.bfloat16)
    logits = jnp.dot(xh, gh, preferred_element_type=f32) + gb_ref[...]
    m = jnp.max(logits, axis=-1, keepdims=True)
    ex = jnp.exp(logits - m)
    w = ex / jnp.sum(ex, axis=-1, keepdims=True)
    lane = jax.lax.broadcasted_iota(jnp.int32, w.shape, 1)
    m1 = jnp.max(w, axis=-1, keepdims=True)
    i1 = jnp.min(jnp.where(w == m1, lane, E), axis=-1, keepdims=True)
    sel1 = lane == i1
    w_rest = jnp.where(sel1, -1.0, w)
    m2 = jnp.max(w_rest, axis=-1, keepdims=True)
    i2 = jnp.min(jnp.where(w_rest == m2, lane, E), axis=-1, keepdims=True)
    sel2 = lane == i2

    oh1 = jnp.where(sel1, 1.0, 0.0)
    oh2 = jnp.where(sel2, 1.0, 0.0)
    r_iota = jax.lax.broadcasted_iota(jnp.int32, (S, S), 0)
    c_iota = jax.lax.broadcasted_iota(jnp.int32, (S, S), 1)
    Ls = jnp.where(c_iota < r_iota, 1.0, 0.0).astype(jnp.bfloat16)
    cum1 = jnp.dot(Ls, oh1.astype(jnp.bfloat16), preferred_element_type=f32)
    cum2 = jnp.dot(Ls, oh2.astype(jnp.bfloat16), preferred_element_type=f32)
    r1 = jnp.sum(cum1 * oh1, axis=-1, keepdims=True)
    r2 = jnp.sum(cum2 * oh2, axis=-1, keepdims=True)
    cnt1 = jnp.sum(oh1, axis=0, keepdims=True)
    cnt2 = jnp.sum(oh2, axis=0, keepdims=True)
    cap = jnp.floor((cnt1 + cnt2 + (T - 1)) / T) * T

    lane8 = jax.lax.broadcasted_iota(jnp.int32, (1, E), 1)
    off = jnp.zeros((1, E), f32)
    for j in range(1, E):
        cap_j = jnp.sum(jnp.where(lane8 == j - 1, cap, 0.0), axis=-1,
                        keepdims=True)
        off = off + jnp.where(lane8 >= j, cap_j, 0.0)

    pos1 = jnp.sum(oh1 * off, axis=-1, keepdims=True) + r1
    pos2 = (jnp.sum(oh2 * (off + cnt1), axis=-1, keepdims=True) + r2)

    lane32 = jax.lax.broadcasted_iota(jnp.int32, (1, 32), 1)
    tstart = (lane32 * T).astype(f32)
    te = jnp.zeros((1, 32), f32)
    for e in range(E):
        off_e = jnp.sum(jnp.where(lane8 == e, off, 0.0), axis=-1,
                        keepdims=True)
        cap_e = jnp.sum(jnp.where(lane8 == e, cap, 0.0), axis=-1,
                        keepdims=True)
        inside = (tstart >= off_e) & (tstart < off_e + cap_e)
        te = te + jnp.where(inside, float(e), 0.0)

    w_out[...] = jnp.concatenate([m1, m2], axis=1)
    pos_out[...] = jnp.concatenate([pos1, pos2], axis=1)
    te_out[...] = te
    xp_out[...] = x.reshape(S, 8, 128)


def _k0_call(x2d, gate_W, gate_b2d):
    return pl.pallas_call(
        _k0_body,
        grid=(1,),
        in_specs=[
            pl.BlockSpec((S, D), lambda i: (0, 0)),
            pl.BlockSpec((D, E), lambda i: (0, 0)),
            pl.BlockSpec((1, E), lambda i: (0, 0)),
        ],
        out_specs=[
            pl.BlockSpec((S, K), lambda i: (0, 0)),
            pl.BlockSpec((S, K), lambda i: (0, 0)),
            pl.BlockSpec((1, 32), lambda i: (0, 0)),
            pl.BlockSpec((S, 8, 128), lambda i: (0, 0, 0)),
        ],
        out_shape=[
            jax.ShapeDtypeStruct((S, K), jnp.float32),
            jax.ShapeDtypeStruct((S, K), jnp.float32),
            jax.ShapeDtypeStruct((1, 32), jnp.float32),
            jax.ShapeDtypeStruct((S, 8, 128), jnp.float32),
        ],
    )(x2d, gate_W, gate_b2d)


def _sc1_body(xp_hbm, p1_hbm, p2_hbm, xs_hbm, rows_v, p1_v, p2_v, sem1,
              sem2):
    wid = lax.axis_index("s") * NSC + lax.axis_index("c")
    base = wid * TPW
    pltpu.sync_copy(xp_hbm.at[pl.ds(base, TPW)], rows_v)
    pltpu.sync_copy(p1_hbm.at[wid], p1_v)
    pltpu.sync_copy(p2_hbm.at[wid], p2_v)
    c1 = pltpu.async_copy(rows_v, xs_hbm.at[p1_v], sem1)
    c2 = pltpu.async_copy(rows_v, xs_hbm.at[p2_v], sem2)
    c1.wait()
    c2.wait()


def _sc1_call(xp3d, pos1, pos2):
    mesh = plsc.VectorSubcoreMesh(core_axis_name="c", subcore_axis_name="s")
    kern = functools.partial(
        pl.kernel, mesh=mesh,
        out_type=jax.ShapeDtypeStruct((NROWS, 8, 128), jnp.float32),
        scratch_types=[
            pltpu.VMEM((TPW, 8, 128), jnp.float32),
            pltpu.VMEM((TPW,), jnp.int32),
            pltpu.VMEM((TPW,), jnp.int32),
            pltpu.SemaphoreType.DMA,
            pltpu.SemaphoreType.DMA,
        ],
    )(_sc1_body)
    return kern(xp3d, pos1, pos2)


def _k2_body(te_ref, xs_ref, w1_ref, b1_ref, w2_ref, b2_ref, y_ref):
    xb = xs_ref[...].reshape(T, D).astype(jnp.bfloat16)
    FC = F // 4
    y = b2_ref[0]
    for f in range(4):
        fs = pl.ds(f * FC, FC)
        h = jnp.dot(xb, w1_ref[0, :, fs], preferred_element_type=jnp.float32)
        h = jnp.maximum(h + b1_ref[0, :, fs], 0.0).astype(jnp.bfloat16)
        y = y + jnp.dot(h, w2_ref[0, fs, :],
                        preferred_element_type=jnp.float32)
    y_ref[...] = y.reshape(T, 8, 128)


def _k2_call(te, xs3d, rW1, rb1, rW2, rb2):
    grid_spec = pltpu.PrefetchScalarGridSpec(
        num_scalar_prefetch=1,
        grid=(NT,),
        in_specs=[
            pl.BlockSpec((T, 8, 128), lambda t, te: (t, 0, 0)),
            pl.BlockSpec((1, D, F), lambda t, te: (te[t], 0, 0)),
            pl.BlockSpec((1, 1, F), lambda t, te: (te[t], 0, 0)),
            pl.BlockSpec((1, F, O), lambda t, te: (te[t], 0, 0)),
            pl.BlockSpec((1, 1, O), lambda t, te: (te[t], 0, 0)),
        ],
        out_specs=pl.BlockSpec((T, 8, 128), lambda t, te: (t, 0, 0)),
    )
    return pl.pallas_call(
        _k2_body,
        grid_spec=grid_spec,
        out_shape=jax.ShapeDtypeStruct((NROWS, 8, 128), jnp.float32),
    )(te, xs3d, rW1, rb1, rW2, rb2)


_CH = 32


def _sc2_body(ys_hbm, p1_hbm, p2_hbm, w1_hbm, w2_hbm, out_hbm,
              y1_v, y2_v, p1_v, p2_v, w1_s, w2_s, sem1, sem2):
    wid = lax.axis_index("s") * NSC + lax.axis_index("c")
    base = wid * TPW
    pltpu.sync_copy(p1_hbm.at[wid], p1_v)
    pltpu.sync_copy(p2_hbm.at[wid], p2_v)
    pltpu.sync_copy(w1_hbm.at[wid], w1_s)
    pltpu.sync_copy(w2_hbm.at[wid], w2_s)
    for c in range(TPW // _CH):
        rows = pl.ds(base + c * _CH, _CH)
        c1 = pltpu.async_copy(ys_hbm.at[p1_v.at[pl.ds(c * _CH, _CH)]], y1_v,
                              sem1)
        c2 = pltpu.async_copy(ys_hbm.at[p2_v.at[pl.ds(c * _CH, _CH)]], y2_v,
                              sem2)
        c1.wait()
        c2.wait()

        def grp(g, _):
            w1v = w1_s[pl.ds(c * _CH + g * 16, 16)]
            w2v = w2_s[pl.ds(c * _CH + g * 16, 16)]
            for i in range(16):
                w1 = w1v[i]
                w2 = w2v[i]
                tok = g * 16 + i

                def sub(j, _, tok=tok, w1=w1, w2=w2):
                    for l in range(8):
                        sl = pl.ds(l * 16, 16)
                        y1_v[tok, j, sl] = (w1 * y1_v[tok, j, sl]
                                            + w2 * y2_v[tok, j, sl])
                    return 0

                lax.fori_loop(0, 8, sub, 0, unroll=False)
            return 0

        lax.fori_loop(0, _CH // 16, grp, 0, unroll=False)
        pltpu.sync_copy(y1_v, out_hbm.at[rows])


def _sc2_call(ys3d, pos1, pos2, w1, w2):
    mesh = plsc.VectorSubcoreMesh(core_axis_name="c", subcore_axis_name="s")
    kern = functools.partial(
        pl.kernel, mesh=mesh,
        out_type=jax.ShapeDtypeStruct((S, 8, 128), jnp.float32),
        scratch_types=[
            pltpu.VMEM((_CH, 8, 128), jnp.float32),
            pltpu.VMEM((_CH, 8, 128), jnp.float32),
            pltpu.VMEM((TPW,), jnp.int32),
            pltpu.VMEM((TPW,), jnp.int32),
            pltpu.VMEM((TPW,), jnp.float32),
            pltpu.VMEM((TPW,), jnp.float32),
            pltpu.SemaphoreType.DMA,
            pltpu.SemaphoreType.DMA,
        ],
    )(_sc2_body)
    return kern(ys3d, pos1, pos2, w1, w2)


def _k1_body(x_ref, r3_ref, w1_ref, b1_ref, w2_ref, b2_ref, out_ref, *,
             tile, n_shared):
    e = pl.program_id(0)
    t = pl.program_id(1)
    rows = pl.ds(t * tile, tile)
    xb = x_ref[...].astype(jnp.bfloat16)
    FC = F // 4
    y = b2_ref[0]
    for f in range(4):
        fs = pl.ds(f * FC, FC)
        h = jnp.dot(xb, w1_ref[0, :, fs], preferred_element_type=jnp.float32)
        h = jnp.maximum(h + b1_ref[0, :, fs], 0.0).astype(jnp.bfloat16)
        y = y + jnp.dot(h, w2_ref[0, fs, :],
                        preferred_element_type=jnp.float32)
    y = y * (1.0 / n_shared)

    @pl.when(e == 0)
    def _init():
        out_ref[rows, :] = y + r3_ref[...].reshape(tile, O)

    @pl.when(e > 0)
    def _accum():
        out_ref[rows, :] += y


def _k1_call(x2d, routed3, sW1, sb1, sW2, sb2, tile=512):
    ns = sW1.shape[0]
    nt = S // tile
    body = functools.partial(_k1_body, tile=tile, n_shared=ns)
    return pl.pallas_call(
        body,
        grid=(ns, nt),
        in_specs=[
            pl.BlockSpec((tile, D), lambda e, t: (t, 0)),
            pl.BlockSpec((tile, 8, 128),
                         lambda e, t: (jnp.where(e == 0, t, 0), 0, 0)),
            pl.BlockSpec((1, D, F), lambda e, t: (e, 0, 0)),
            pl.BlockSpec((1, 1, F), lambda e, t: (e, 0, 0)),
            pl.BlockSpec((1, F, O), lambda e, t: (e, 0, 0)),
            pl.BlockSpec((1, 1, O), lambda e, t: (e, 0, 0)),
        ],
        out_specs=pl.BlockSpec((S, O), lambda e, t: (0, 0)),
        out_shape=jax.ShapeDtypeStruct((S, O), jnp.float32),
    )(x2d, routed3, sW1, sb1, sW2, sb2)


def kernel(x, gate_W, gate_b, shared_W1, shared_b1, shared_W2, shared_b2,
           routed_W1, routed_b1, routed_W2, routed_b2):
    B = x.shape[0]
    x2d = x.reshape(S, D)

    w_out, pos_out, te_out, xp = _k0_call(x2d, gate_W, gate_b.reshape(1, E))

    te = te_out.reshape(32).astype(jnp.int32)
    posT = pos_out.T.astype(jnp.int32).reshape(K, NW, TPW)
    wT = w_out.T.reshape(K, NW, TPW)

    xs = _sc1_call(xp, posT[0], posT[1])

    ys = _k2_call(te, xs,
                  routed_W1.astype(jnp.bfloat16), routed_b1.reshape(E, 1, F),
                  routed_W2.astype(jnp.bfloat16), routed_b2.reshape(E, 1, O))

    routed3 = _sc2_call(ys, posT[0], posT[1], wT[0], wT[1])

    out = _k1_call(x2d, routed3, shared_W1.astype(jnp.bfloat16),
                   shared_b1.reshape(-1, 1, F),
                   shared_W2.astype(jnp.bfloat16),
                   shared_b2.reshape(-1, 1, O))
    return out.reshape(B, S, O)


# speedup vs baseline: 1.8996x; 1.0527x over previous
import functools

import jax
import jax.numpy as jnp
from jax import lax
from jax.experimental import pallas as pl
from jax.experimental.pallas import tpu as pltpu
from jax.experimental.pallas import tpu_sc as plsc

S, D, F, O, E, K, T = 2048, 1024, 2048, 1024, 8, 2, 256
NT = S * K // T + E
NROWS = NT * T
NSC, NSUB = 2, 16
NW = NSC * NSUB
TPW = S // NW


def _k0_body(x_ref, gw_ref, gb_ref, w_out, pos_out, te_out, xp_out):
    x = x_ref[...]
    f32 = jnp.float32
    xh = x.astype(jnp.bfloat16)
    gh = gw_ref[...].astype(jnp.bfloat16)
    logits = jnp.dot(xh, gh, preferred_element_type=f32) + gb_ref[...]
    m = jnp.max(logits, axis=-1, keepdims=True)
    ex = jnp.exp(logits - m)
    w = ex / jnp.sum(ex, axis=-1, keepdims=True)
    lane = jax.lax.broadcasted_iota(jnp.int32, w.shape, 1)
    m1 = jnp.max(w, axis=-1, keepdims=True)
    i1 = jnp.min(jnp.where(w == m1, lane, E), axis=-1, keepdims=True)
    sel1 = lane == i1
    w_rest = jnp.where(sel1, -1.0, w)
    m2 = jnp.max(w_rest, axis=-1, keepdims=True)
    i2 = jnp.min(jnp.where(w_rest == m2, lane, E), axis=-1, keepdims=True)
    sel2 = lane == i2

    oh1 = jnp.where(sel1, 1.0, 0.0)
    oh2 = jnp.where(sel2, 1.0, 0.0)
    r_iota = jax.lax.broadcasted_iota(jnp.int32, (S, S), 0)
    c_iota = jax.lax.broadcasted_iota(jnp.int32, (S, S), 1)
    Ls = jnp.where(c_iota < r_iota, 1.0, 0.0).astype(jnp.bfloat16)
    cum1 = jnp.dot(Ls, oh1.astype(jnp.bfloat16), preferred_element_type=f32)
    cum2 = jnp.dot(Ls, oh2.astype(jnp.bfloat16), preferred_element_type=f32)
    r1 = jnp.sum(cum1 * oh1, axis=-1, keepdims=True)
    r2 = jnp.sum(cum2 * oh2, axis=-1, keepdims=True)
    cnt1 = jnp.sum(oh1, axis=0, keepdims=True)
    cnt2 = jnp.sum(oh2, axis=0, keepdims=True)
    cap = jnp.floor((cnt1 + cnt2 + (T - 1)) / T) * T

    lane8 = jax.lax.broadcasted_iota(jnp.int32, (1, E), 1)
    off = jnp.zeros((1, E), f32)
    for j in range(1, E):
        cap_j = jnp.sum(jnp.where(lane8 == j - 1, cap, 0.0), axis=-1,
                        keepdims=True)
        off = off + jnp.where(lane8 >= j, cap_j, 0.0)

    pos1 = jnp.sum(oh1 * off, axis=-1, keepdims=True) + r1
    pos2 = (jnp.sum(oh2 * (off + cnt1), axis=-1, keepdims=True) + r2)

    lane32 = jax.lax.broadcasted_iota(jnp.int32, (1, 32), 1)
    tstart = (lane32 * T).astype(f32)
    te = jnp.zeros((1, 32), f32)
    for e in range(E):
        off_e = jnp.sum(jnp.where(lane8 == e, off, 0.0), axis=-1,
                        keepdims=True)
        cap_e = jnp.sum(jnp.where(lane8 == e, cap, 0.0), axis=-1,
                        keepdims=True)
        inside = (tstart >= off_e) & (tstart < off_e + cap_e)
        te = te + jnp.where(inside, float(e), 0.0)

    w_out[...] = jnp.concatenate([m1, m2], axis=1)
    pos_out[...] = jnp.concatenate([pos1, pos2], axis=1)
    te_out[...] = te
    xp_out[...] = x.reshape(S, 8, 128)


def _k0_call(x2d, gate_W, gate_b2d):
    return pl.pallas_call(
        _k0_body,
        grid=(1,),
        in_specs=[
            pl.BlockSpec((S, D), lambda i: (0, 0)),
            pl.BlockSpec((D, E), lambda i: (0, 0)),
            pl.BlockSpec((1, E), lambda i: (0, 0)),
        ],
        out_specs=[
            pl.BlockSpec((S, K), lambda i: (0, 0)),
            pl.BlockSpec((S, K), lambda i: (0, 0)),
            pl.BlockSpec((1, 32), lambda i: (0, 0)),
            pl.BlockSpec((S, 8, 128), lambda i: (0, 0, 0)),
        ],
        out_shape=[
            jax.ShapeDtypeStruct((S, K), jnp.float32),
            jax.ShapeDtypeStruct((S, K), jnp.float32),
            jax.ShapeDtypeStruct((1, 32), jnp.float32),
            jax.ShapeDtypeStruct((S, 8, 128), jnp.float32),
        ],
    )(x2d, gate_W, gate_b2d)


def _sc1_body(xp_hbm, p1_hbm, p2_hbm, xs_hbm, rows_v, p1_v, p2_v, sem1,
              sem2):
    wid = lax.axis_index("s") * NSC + lax.axis_index("c")
    base = wid * TPW
    pltpu.sync_copy(xp_hbm.at[pl.ds(base, TPW)], rows_v)
    pltpu.sync_copy(p1_hbm.at[wid], p1_v)
    pltpu.sync_copy(p2_hbm.at[wid], p2_v)
    c1 = pltpu.async_copy(rows_v, xs_hbm.at[p1_v], sem1)
    c2 = pltpu.async_copy(rows_v, xs_hbm.at[p2_v], sem2)
    c1.wait()
    c2.wait()


def _sc1_call(xp3d, pos1, pos2):
    mesh = plsc.VectorSubcoreMesh(core_axis_name="c", subcore_axis_name="s")
    kern = functools.partial(
        pl.kernel, mesh=mesh,
        out_type=jax.ShapeDtypeStruct((NROWS, 8, 128), jnp.float32),
        scratch_types=[
            pltpu.VMEM((TPW, 8, 128), jnp.float32),
            pltpu.VMEM((TPW,), jnp.int32),
            pltpu.VMEM((TPW,), jnp.int32),
            pltpu.SemaphoreType.DMA,
            pltpu.SemaphoreType.DMA,
        ],
    )(_sc1_body)
    return kern(xp3d, pos1, pos2)


def _k2_body(te_ref, xs_ref, w1_ref, b1_ref, w2_ref, b2_ref, y_ref):
    xb = xs_ref[...].reshape(T, D).astype(jnp.bfloat16)
    FC = F // 4
    y = b2_ref[0]
    for f in range(4):
        fs = pl.ds(f * FC, FC)
        h = jnp.dot(xb, w1_ref[0, :, fs], preferred_element_type=jnp.float32)
        h = jnp.maximum(h + b1_ref[0, :, fs], 0.0).astype(jnp.bfloat16)
        y = y + jnp.dot(h, w2_ref[0, fs, :],
                        preferred_element_type=jnp.float32)
    y_ref[...] = y.reshape(T, 8, 128)


def _k2_call(te, xs3d, rW1, rb1, rW2, rb2):
    grid_spec = pltpu.PrefetchScalarGridSpec(
        num_scalar_prefetch=1,
        grid=(NT,),
        in_specs=[
            pl.BlockSpec((T, 8, 128), lambda t, te: (t, 0, 0)),
            pl.BlockSpec((1, D, F), lambda t, te: (te[t], 0, 0)),
            pl.BlockSpec((1, 1, F), lambda t, te: (te[t], 0, 0)),
            pl.BlockSpec((1, F, O), lambda t, te: (te[t], 0, 0)),
            pl.BlockSpec((1, 1, O), lambda t, te: (te[t], 0, 0)),
        ],
        out_specs=pl.BlockSpec((T, 8, 128), lambda t, te: (t, 0, 0)),
    )
    return pl.pallas_call(
        _k2_body,
        grid_spec=grid_spec,
        out_shape=jax.ShapeDtypeStruct((NROWS, 8, 128), jnp.float32),
    )(te, xs3d, rW1, rb1, rW2, rb2)


_CH = 16


def _sc2_body(ys_hbm, p1_hbm, p2_hbm, w1_hbm, w2_hbm, out_hbm,
              y1a, y2a, y1b, y2b, p1_v, p2_v, w1_s, w2_s, sem1, sem2):
    wid = lax.axis_index("s") * NSC + lax.axis_index("c")
    base = wid * TPW
    pltpu.sync_copy(p1_hbm.at[wid], p1_v)
    pltpu.sync_copy(p2_hbm.at[wid], p2_v)
    pltpu.sync_copy(w1_hbm.at[wid], w1_s)
    pltpu.sync_copy(w2_hbm.at[wid], w2_s)
    nch = TPW // _CH
    bufs = [(y1a, y2a), (y1b, y2b)]

    def issue(c):
        y1, y2 = bufs[c % 2]
        a = pltpu.async_copy(ys_hbm.at[p1_v.at[pl.ds(c * _CH, _CH)]], y1,
                             sem1)
        b = pltpu.async_copy(ys_hbm.at[p2_v.at[pl.ds(c * _CH, _CH)]], y2,
                             sem2)
        return a, b

    pend = issue(0)
    for c in range(nch):
        y1_v, y2_v = bufs[c % 2]
        pend[0].wait()
        pend[1].wait()
        if c + 1 < nch:
            pend = issue(c + 1)
        w1v = w1_s[pl.ds(c * _CH, 16)]
        w2v = w2_s[pl.ds(c * _CH, 16)]
        for i in range(16):
            w1 = w1v[i]
            w2 = w2v[i]

            def sub(j, _, tok=i, w1=w1, w2=w2, y1_v=y1_v, y2_v=y2_v):
                for l in range(8):
                    sl = pl.ds(l * 16, 16)
                    y1_v[tok, j, sl] = (w1 * y1_v[tok, j, sl]
                                        + w2 * y2_v[tok, j, sl])
                return 0

            lax.fori_loop(0, 8, sub, 0, unroll=False)
        pltpu.sync_copy(y1_v, out_hbm.at[pl.ds(base + c * _CH, _CH)])


def _sc2_call(ys3d, pos1, pos2, w1, w2):
    mesh = plsc.VectorSubcoreMesh(core_axis_name="c", subcore_axis_name="s")
    kern = functools.partial(
        pl.kernel, mesh=mesh,
        out_type=jax.ShapeDtypeStruct((S, 8, 128), jnp.float32),
        scratch_types=[
            pltpu.VMEM((_CH, 8, 128), jnp.float32),
            pltpu.VMEM((_CH, 8, 128), jnp.float32),
            pltpu.VMEM((_CH, 8, 128), jnp.float32),
            pltpu.VMEM((_CH, 8, 128), jnp.float32),
            pltpu.VMEM((TPW,), jnp.int32),
            pltpu.VMEM((TPW,), jnp.int32),
            pltpu.VMEM((TPW,), jnp.float32),
            pltpu.VMEM((TPW,), jnp.float32),
            pltpu.SemaphoreType.DMA,
            pltpu.SemaphoreType.DMA,
        ],
    )(_sc2_body)
    return kern(ys3d, pos1, pos2, w1, w2)


def _k1a_body(x_ref, w1_ref, b1_ref, w2_ref, b2_ref, out_ref, *, tile,
              n_shared):
    e = pl.program_id(0)
    t = pl.program_id(1)
    rows = pl.ds(t * tile, tile)
    xb = x_ref[...].astype(jnp.bfloat16)
    FC = F // 4
    y = b2_ref[0]
    for f in range(4):
        fs = pl.ds(f * FC, FC)
        h = jnp.dot(xb, w1_ref[0, :, fs], preferred_element_type=jnp.float32)
        h = jnp.maximum(h + b1_ref[0, :, fs], 0.0).astype(jnp.bfloat16)
        y = y + jnp.dot(h, w2_ref[0, fs, :],
                        preferred_element_type=jnp.float32)
    y = y * (1.0 / n_shared)

    @pl.when(e == 0)
    def _init():
        out_ref[rows, :] = y

    @pl.when(e > 0)
    def _accum():
        out_ref[rows, :] += y


def _k1a_call(x2d, sW1, sb1, sW2, sb2, tile=512):
    ns = sW1.shape[0]
    nt = S // tile
    body = functools.partial(_k1a_body, tile=tile, n_shared=ns)
    return pl.pallas_call(
        body,
        grid=(ns, nt),
        in_specs=[
            pl.BlockSpec((tile, D), lambda e, t: (t, 0)),
            pl.BlockSpec((1, D, F), lambda e, t: (e, 0, 0)),
            pl.BlockSpec((1, 1, F), lambda e, t: (e, 0, 0)),
            pl.BlockSpec((1, F, O), lambda e, t: (e, 0, 0)),
            pl.BlockSpec((1, 1, O), lambda e, t: (e, 0, 0)),
        ],
        out_specs=pl.BlockSpec((S, O), lambda e, t: (0, 0)),
        out_shape=jax.ShapeDtypeStruct((S, O), jnp.float32),
    )(x2d, sW1, sb1, sW2, sb2)


def _k1b_body(sh_ref, r3_ref, out_ref, *, tile):
    out_ref[...] = sh_ref[...] + r3_ref[...].reshape(tile, O)


def _k1b_call(shared2d, routed3, tile=1024):
    nt = S // tile
    body = functools.partial(_k1b_body, tile=tile)
    return pl.pallas_call(
        body,
        grid=(nt,),
        in_specs=[
            pl.BlockSpec((tile, O), lambda t: (t, 0)),
            pl.BlockSpec((tile, 8, 128), lambda t: (t, 0, 0)),
        ],
        out_specs=pl.BlockSpec((tile, O), lambda t: (t, 0)),
        out_shape=jax.ShapeDtypeStruct((S, O), jnp.float32),
    )(shared2d, routed3)


def kernel(x, gate_W, gate_b, shared_W1, shared_b1, shared_W2, shared_b2,
           routed_W1, routed_b1, routed_W2, routed_b2):
    B = x.shape[0]
    x2d = x.reshape(S, D)

    w_out, pos_out, te_out, xp = _k0_call(x2d, gate_W, gate_b.reshape(1, E))

    te = te_out.reshape(32).astype(jnp.int32)
    posT = pos_out.T.astype(jnp.int32).reshape(K, NW, TPW)
    wT = w_out.T.reshape(K, NW, TPW)

    xs = _sc1_call(xp, posT[0], posT[1])

    shared = _k1a_call(x2d, shared_W1.astype(jnp.bfloat16),
                       shared_b1.reshape(-1, 1, F),
                       shared_W2.astype(jnp.bfloat16),
                       shared_b2.reshape(-1, 1, O))

    ys = _k2_call(te, xs,
                  routed_W1.astype(jnp.bfloat16), routed_b1.reshape(E, 1, F),
                  routed_W2.astype(jnp.bfloat16), routed_b2.reshape(E, 1, O))

    routed3 = _sc2_call(ys, posT[0], posT[1], wT[0], wT[1])

    out = _k1b_call(shared, routed3)
    return out.reshape(B, S, O)
